# Initial kernel scaffold; baseline (speedup 1.0000x reference)
import jax
import jax.numpy as jnp
from jax.experimental import pallas as pl


def kernel(centers, levels, weights, w2c, n_samples, initial_size):
    raise NotImplementedError("write your pallas kernel here")


# speedup vs baseline: 6.1245x; 6.1245x over previous
import functools

import jax
import jax.numpy as jnp
from jax import lax
from jax.experimental import pallas as pl
from jax.experimental.pallas import tpu as pltpu
from jax.experimental.pallas import tpu_sc as plsc

N = 200000
NS = 131072
ROWS = 1568
VALID_ROWS = 1563
NT = 12500
FX = 1000.0
FY = 1000.0
WIDTH = 1600.0
HEIGHT = 1200.0
NEAR = 0.1
FAR = 100.0
SEED = 42

NWORK = 32
QPW = NS // NWORK
CH = 2048
NGRP = CH // 16
NK = CH // 128


def _k1_body(ct_ref, w_ref, m_ref, out_ref):
    xb = ct_ref[0, :].astype(jnp.bfloat16).astype(jnp.float32)
    yb = ct_ref[1, :].astype(jnp.bfloat16).astype(jnp.float32)
    zb = ct_ref[2, :].astype(jnp.bfloat16).astype(jnp.float32)

    def row(j):
        return ((xb * m_ref[j, 0] + yb * m_ref[j, 1]) + zb * m_ref[j, 2]) + m_ref[j, 3]

    x = row(0)
    y = row(1)
    z = row(2)
    zc = jnp.maximum(z, 1e-6)
    u_img = x / zc * FX
    v_img = y / zc * FY
    vis = ((z > NEAR) & (z < FAR)
           & (jnp.abs(u_img) <= WIDTH * 0.5)
           & (jnp.abs(v_img) <= HEIGHT * 0.5))
    out_ref[...] = w_ref[...] * vis.astype(jnp.float32)


def _k2_body(w_ref, out_ref):
    def step(i, acc):
        return acc + w_ref[pl.ds(i * 8, 8), :]

    acc = lax.fori_loop(0, ROWS // 8, step, jnp.zeros((8, 128), jnp.float32))
    acc = acc + pltpu.roll(acc, 4, 0)
    acc = acc + pltpu.roll(acc, 2, 0)
    acc = acc + pltpu.roll(acc, 1, 0)
    out_ref[0] = jnp.sum(acc[0, :]) + 1e-12


def _k3_body(wt_ref, tot_ref, s1_ref):
    t = tot_ref[0]
    acc = wt_ref[0:1, :] / t
    s1_ref[0:1, :] = acc
    for c in range(1, 128):
        acc = acc + wt_ref[c:c + 1, :] / t
        s1_ref[c:c + 1, :] = acc


def _k5_body(s1_ref, off_ref, w_ref, tot_ref, cdf_ref, p_ref):
    p_ref[...] = w_ref[...] / tot_ref[0]
    cdf_ref[...] = s1_ref[...] + off_ref[...]


def _sc_body(table_h, rows_h, u_h, packed_h, nxh, nyh, nzh, isz_h,
             px_h, py_h, pz_h, pr_h,
             tab_v, u_v, rc_v, fine_v, idx_v, rows_v,
             nx_v, ny_v, nz_v, px_v, py_v, pz_v, pr_v, isz_v, sem):
    wid = lax.axis_index("s") * 2 + lax.axis_index("c")
    pltpu.sync_copy(table_h, tab_v)
    pltpu.sync_copy(isz_h, isz_v)
    iota16 = lax.iota(jnp.int32, 16)

    def half_body(h, carry):
        base = wid * QPW + h * CH
        pltpu.sync_copy(u_h.at[pl.ds(base, CH)], u_v)
        pltpu.sync_copy(nxh.at[pl.ds(base, CH)], nx_v)
        pltpu.sync_copy(nyh.at[pl.ds(base, CH)], ny_v)
        pltpu.sync_copy(nzh.at[pl.ds(base, CH)], nz_v)

        def coarse(g, c):
            u16 = u_v[pl.ds(g * 16, 16)]
            pos = jnp.zeros((16,), jnp.int32)
            for stp in (8192, 4096, 2048, 1024, 512, 256, 128,
                        64, 32, 16, 8, 4, 2, 1):
                cand = pos + stp
                inb = cand <= NT
                gidx = jnp.minimum(cand, NT) - 1
                tv = plsc.load_gather(tab_v, [gidx])
                pos = jnp.where(inb & (tv < u16), cand, pos)
            rcl = jnp.minimum(pos, NT - 1)
            rc_v[g // 8, pl.ds((g % 8) * 16, 16)] = rcl
            return c

        lax.fori_loop(0, NGRP, coarse, 0)

        cps = [pltpu.async_copy(rows_h.at[rc_v.at[k]],
                                fine_v.at[pl.ds(k * 128, 128), :], sem)
               for k in range(NK)]
        for c in cps:
            c.wait()

        def fine(g, c):
            u16 = u_v[pl.ds(g * 16, 16)]
            r16 = rc_v[g // 8, pl.ds((g % 8) * 16, 16)]
            qidx = g * 16 + iota16
            pos = jnp.zeros((16,), jnp.int32)
            for stp in (16, 8, 4, 2, 1):
                cand = pos + stp
                inb = cand <= 16
                gidx = jnp.minimum(cand, 16) - 1
                tv = plsc.load_gather(fine_v, [qidx, gidx])
                pos = jnp.where(inb & (tv < u16), cand, pos)
            idx16 = jnp.minimum(r16 * 16 + pos, N - 1)
            idx_v[g // 8, pl.ds((g % 8) * 16, 16)] = idx16
            return c

        lax.fori_loop(0, NGRP, fine, 0)

        cps = [pltpu.async_copy(packed_h.at[idx_v.at[k]],
                                rows_v.at[pl.ds(k * 128, 128), :], sem)
               for k in range(NK)]
        for c in cps:
            c.wait()

        def combine(g, c):
            q0 = g * 16
            qidx = q0 + iota16
            zero = jnp.zeros((16,), jnp.int32)
            cx = plsc.load_gather(rows_v, [qidx, zero])
            cy = plsc.load_gather(rows_v, [qidx, zero + 1])
            cz = plsc.load_gather(rows_v, [qidx, zero + 2])
            pp = plsc.load_gather(rows_v, [qidx, zero + 3])
            sz = plsc.load_gather(rows_v, [qidx, zero + 4])
            size = sz * isz_v[...]
            px_v[pl.ds(q0, 16)] = cx + nx_v[pl.ds(q0, 16)] * size
            py_v[pl.ds(q0, 16)] = cy + ny_v[pl.ds(q0, 16)] * size
            pz_v[pl.ds(q0, 16)] = cz + nz_v[pl.ds(q0, 16)] * size
            pr_v[pl.ds(q0, 16)] = pp
            return c

        lax.fori_loop(0, NGRP, combine, 0)

        pltpu.sync_copy(px_v, px_h.at[pl.ds(base, CH)])
        pltpu.sync_copy(py_v, py_h.at[pl.ds(base, CH)])
        pltpu.sync_copy(pz_v, pz_h.at[pl.ds(base, CH)])
        pltpu.sync_copy(pr_v, pr_h.at[pl.ds(base, CH)])
        return carry

    lax.fori_loop(0, QPW // CH, half_body, 0)


def _make_sc_kernel():
    mesh = plsc.VectorSubcoreMesh(core_axis_name="c", subcore_axis_name="s")
    out1 = jax.ShapeDtypeStruct((NS,), jnp.float32)
    return pl.kernel(
        _sc_body,
        mesh=mesh,
        compiler_params=pltpu.CompilerParams(needs_layout_passes=False, use_tc_tiling_on_sc=False),
        out_type=[out1, out1, out1, out1],
        scratch_types=[
            pltpu.VMEM((NT,), jnp.float32),
            pltpu.VMEM((CH,), jnp.float32),
            pltpu.VMEM((NK, 128), jnp.int32),
            pltpu.VMEM((CH, 16), jnp.float32),
            pltpu.VMEM((NK, 128), jnp.int32),
            pltpu.VMEM((CH, 16), jnp.float32),
            pltpu.VMEM((CH,), jnp.float32),
            pltpu.VMEM((CH,), jnp.float32),
            pltpu.VMEM((CH,), jnp.float32),
            pltpu.VMEM((CH,), jnp.float32),
            pltpu.VMEM((CH,), jnp.float32),
            pltpu.VMEM((CH,), jnp.float32),
            pltpu.VMEM((CH,), jnp.float32),
            pltpu.VMEM((16,), jnp.float32),
            pltpu.SemaphoreType.DMA,
        ],
    )


def kernel(centers, levels, weights, w2c, n_samples, initial_size):
    ct = centers.T
    mb = w2c.astype(jnp.bfloat16).astype(jnp.float32)
    w_eff = pl.pallas_call(
        _k1_body,
        out_shape=jax.ShapeDtypeStruct((N,), jnp.float32),
        in_specs=[pl.BlockSpec((3, N), lambda: (0, 0)),
                  pl.BlockSpec((N,), lambda: (0,)),
                  pl.BlockSpec(memory_space=pltpu.SMEM)],
        out_specs=pl.BlockSpec((N,), lambda: (0,)),
    )(ct, weights, mb)

    wp = jnp.pad(w_eff, (0, ROWS * 128 - N)).reshape(ROWS, 128)
    total = pl.pallas_call(
        _k2_body,
        out_shape=jax.ShapeDtypeStruct((1,), jnp.float32),
        in_specs=[pl.BlockSpec((ROWS, 128), lambda: (0, 0))],
        out_specs=pl.BlockSpec(memory_space=pltpu.SMEM),
    )(wp)

    wt = wp.T
    s1t = pl.pallas_call(
        _k3_body,
        out_shape=jax.ShapeDtypeStruct((128, ROWS), jnp.float32),
        in_specs=[pl.BlockSpec((128, ROWS), lambda: (0, 0)),
                  pl.BlockSpec((1,), lambda: (0,))],
        out_specs=pl.BlockSpec((128, ROWS), lambda: (0, 0)),
    )(wt, total)

    tot1 = s1t[127, :VALID_ROWS]
    lvl2 = jnp.cumsum(tot1)
    off1 = jnp.pad(jnp.concatenate([jnp.zeros((1,), jnp.float32), lvl2[:-1]]),
                   (0, ROWS - VALID_ROWS)).reshape(1, ROWS)

    cdft, probs = pl.pallas_call(
        _k5_body,
        out_shape=[jax.ShapeDtypeStruct((128, ROWS), jnp.float32),
                   jax.ShapeDtypeStruct((N,), jnp.float32)],
        in_specs=[pl.BlockSpec((128, ROWS), lambda: (0, 0)),
                  pl.BlockSpec((1, ROWS), lambda: (0, 0)),
                  pl.BlockSpec((N,), lambda: (0,)),
                  pl.BlockSpec((1,), lambda: (0,))],
        out_specs=[pl.BlockSpec((128, ROWS), lambda: (0, 0)),
                   pl.BlockSpec((N,), lambda: (0,))],
    )(s1t, off1, w_eff, total)

    cdf = cdft.T.reshape(-1)[:N]
    table = cdf[15::16]
    cdf_rows = cdf.reshape(NT, 16)
    sz_bits = lax.bitcast_convert_type((jnp.int32(127) - levels) << 23,
                                       jnp.float32)
    packed = jnp.concatenate(
        [centers, probs[:, None], sz_bits[:, None],
         jnp.zeros((N, 11), jnp.float32)], axis=1)

    key = jax.random.key(SEED)
    ku, kn = jax.random.split(key)
    u = jax.random.uniform(ku, (NS,), dtype=jnp.float32)
    noise = jax.random.normal(kn, (NS, 3), dtype=jnp.float32)
    isz = jnp.broadcast_to(initial_size, (16,))

    px, py, pz, pr = _make_sc_kernel()(
        table, cdf_rows, u, packed,
        noise[:, 0], noise[:, 1], noise[:, 2], isz)
    return jnp.stack([px, py, pz, pr], axis=1)


# speedup vs baseline: 7.3029x; 1.1924x over previous
import functools

import jax
import jax.numpy as jnp
from jax import lax
from jax.experimental import pallas as pl
from jax.experimental.pallas import tpu as pltpu
from jax.experimental.pallas import tpu_sc as plsc

N = 200000
NS = 131072
ROWS = 1568
VALID_ROWS = 1563
NT = 12500
FX = 1000.0
FY = 1000.0
WIDTH = 1600.0
HEIGHT = 1200.0
NEAR = 0.1
FAR = 100.0
SEED = 42

NWORK = 32
QPW = NS // NWORK
CH = 2048
NGRP = CH // 16
NK = CH // 128


def _k1_body(ct_ref, w_ref, m_ref, out_ref):
    xb = ct_ref[0, :].astype(jnp.bfloat16).astype(jnp.float32)
    yb = ct_ref[1, :].astype(jnp.bfloat16).astype(jnp.float32)
    zb = ct_ref[2, :].astype(jnp.bfloat16).astype(jnp.float32)

    def row(j):
        return ((xb * m_ref[j, 0] + yb * m_ref[j, 1]) + zb * m_ref[j, 2]) + m_ref[j, 3]

    x = row(0)
    y = row(1)
    z = row(2)
    zc = jnp.maximum(z, 1e-6)
    u_img = x / zc * FX
    v_img = y / zc * FY
    vis = ((z > NEAR) & (z < FAR)
           & (jnp.abs(u_img) <= WIDTH * 0.5)
           & (jnp.abs(v_img) <= HEIGHT * 0.5))
    out_ref[...] = w_ref[...] * vis.astype(jnp.float32)


def _k2_body(w_ref, out_ref):
    def step(i, acc):
        return acc + w_ref[pl.ds(i * 8, 8), :]

    acc = lax.fori_loop(0, ROWS // 8, step, jnp.zeros((8, 128), jnp.float32))
    acc = acc + pltpu.roll(acc, 4, 0)
    acc = acc + pltpu.roll(acc, 2, 0)
    acc = acc + pltpu.roll(acc, 1, 0)
    out_ref[0] = jnp.sum(acc[0, :]) + 1e-12


def _k3_body(wt_ref, tot_ref, s1_ref):
    t = tot_ref[0]
    acc = wt_ref[0:1, :] / t
    s1_ref[0:1, :] = acc
    for c in range(1, 128):
        acc = acc + wt_ref[c:c + 1, :] / t
        s1_ref[c:c + 1, :] = acc


def _k5_body(s1_ref, off_ref, w_ref, tot_ref, cdf_ref, p_ref):
    p_ref[...] = w_ref[...] / tot_ref[0]
    cdf_ref[...] = s1_ref[...] + off_ref[...]


def _sc_body(table_h, rows_h, u_h, packed_h, nxh, nyh, nzh, isz_h,
             px_h, py_h, pz_h, pr_h,
             tab_v, u_v, rc_v, fine_v, idx_v, rows_v,
             nx_v, ny_v, nz_v, px_v, py_v, pz_v, pr_v, isz_v, sem):
    wid = lax.axis_index("s") * 2 + lax.axis_index("c")
    pltpu.sync_copy(table_h, tab_v)
    pltpu.sync_copy(isz_h, isz_v)
    iota16 = lax.iota(jnp.int32, 16)

    def half_body(h, carry):
        base = wid * QPW + h * CH
        pltpu.sync_copy(u_h.at[pl.ds(base, CH)], u_v)
        pltpu.sync_copy(nxh.at[pl.ds(base, CH)], nx_v)
        pltpu.sync_copy(nyh.at[pl.ds(base, CH)], ny_v)
        pltpu.sync_copy(nzh.at[pl.ds(base, CH)], nz_v)

        @plsc.parallel_loop(0, NGRP, unroll=4)
        def coarse(g):
            u16 = u_v[pl.ds(g * 16, 16)]
            pos = jnp.zeros((16,), jnp.int32)
            for stp in (8192, 4096, 2048, 1024, 512, 256, 128,
                        64, 32, 16, 8, 4, 2, 1):
                cand = pos + stp
                inb = cand <= NT
                gidx = jnp.minimum(cand, NT) - 1
                tv = plsc.load_gather(tab_v, [gidx])
                pos = jnp.where(inb & (tv < u16), cand, pos)
            rcl = jnp.minimum(pos, NT - 1)
            rc_v[g // 8, pl.ds((g % 8) * 16, 16)] = rcl

        cps = [pltpu.async_copy(rows_h.at[rc_v.at[k]],
                                fine_v.at[pl.ds(k * 128, 128), :], sem)
               for k in range(NK)]
        for c in cps:
            c.wait()

        @plsc.parallel_loop(0, NGRP, unroll=4)
        def fine(g):
            u16 = u_v[pl.ds(g * 16, 16)]
            r16 = rc_v[g // 8, pl.ds((g % 8) * 16, 16)]
            qidx = g * 16 + iota16
            pos = jnp.zeros((16,), jnp.int32)
            for stp in (16, 8, 4, 2, 1):
                cand = pos + stp
                inb = cand <= 16
                gidx = jnp.minimum(cand, 16) - 1
                tv = plsc.load_gather(fine_v, [qidx, gidx])
                pos = jnp.where(inb & (tv < u16), cand, pos)
            idx16 = jnp.minimum(r16 * 16 + pos, N - 1)
            idx_v[g // 8, pl.ds((g % 8) * 16, 16)] = idx16

        cps = [pltpu.async_copy(packed_h.at[idx_v.at[k]],
                                rows_v.at[pl.ds(k * 128, 128), :], sem)
               for k in range(NK)]
        for c in cps:
            c.wait()

        @plsc.parallel_loop(0, NGRP, unroll=4)
        def combine(g):
            q0 = g * 16
            qidx = q0 + iota16
            zero = jnp.zeros((16,), jnp.int32)
            cx = plsc.load_gather(rows_v, [qidx, zero])
            cy = plsc.load_gather(rows_v, [qidx, zero + 1])
            cz = plsc.load_gather(rows_v, [qidx, zero + 2])
            pp = plsc.load_gather(rows_v, [qidx, zero + 3])
            sz = plsc.load_gather(rows_v, [qidx, zero + 4])
            size = sz * isz_v[...]
            px_v[pl.ds(q0, 16)] = cx + nx_v[pl.ds(q0, 16)] * size
            py_v[pl.ds(q0, 16)] = cy + ny_v[pl.ds(q0, 16)] * size
            pz_v[pl.ds(q0, 16)] = cz + nz_v[pl.ds(q0, 16)] * size
            pr_v[pl.ds(q0, 16)] = pp

        pltpu.sync_copy(px_v, px_h.at[pl.ds(base, CH)])
        pltpu.sync_copy(py_v, py_h.at[pl.ds(base, CH)])
        pltpu.sync_copy(pz_v, pz_h.at[pl.ds(base, CH)])
        pltpu.sync_copy(pr_v, pr_h.at[pl.ds(base, CH)])
        return carry

    lax.fori_loop(0, QPW // CH, half_body, 0)


def _make_sc_kernel():
    mesh = plsc.VectorSubcoreMesh(core_axis_name="c", subcore_axis_name="s")
    out1 = jax.ShapeDtypeStruct((NS,), jnp.float32)
    return pl.kernel(
        _sc_body,
        mesh=mesh,
        compiler_params=pltpu.CompilerParams(needs_layout_passes=False, use_tc_tiling_on_sc=False),
        out_type=[out1, out1, out1, out1],
        scratch_types=[
            pltpu.VMEM((NT,), jnp.float32),
            pltpu.VMEM((CH,), jnp.float32),
            pltpu.VMEM((NK, 128), jnp.int32),
            pltpu.VMEM((CH, 16), jnp.float32),
            pltpu.VMEM((NK, 128), jnp.int32),
            pltpu.VMEM((CH, 16), jnp.float32),
            pltpu.VMEM((CH,), jnp.float32),
            pltpu.VMEM((CH,), jnp.float32),
            pltpu.VMEM((CH,), jnp.float32),
            pltpu.VMEM((CH,), jnp.float32),
            pltpu.VMEM((CH,), jnp.float32),
            pltpu.VMEM((CH,), jnp.float32),
            pltpu.VMEM((CH,), jnp.float32),
            pltpu.VMEM((16,), jnp.float32),
            pltpu.SemaphoreType.DMA,
        ],
    )


def kernel(centers, levels, weights, w2c, n_samples, initial_size):
    ct = centers.T
    mb = w2c.astype(jnp.bfloat16).astype(jnp.float32)
    w_eff = pl.pallas_call(
        _k1_body,
        out_shape=jax.ShapeDtypeStruct((N,), jnp.float32),
        in_specs=[pl.BlockSpec((3, N), lambda: (0, 0)),
                  pl.BlockSpec((N,), lambda: (0,)),
                  pl.BlockSpec(memory_space=pltpu.SMEM)],
        out_specs=pl.BlockSpec((N,), lambda: (0,)),
    )(ct, weights, mb)

    wp = jnp.pad(w_eff, (0, ROWS * 128 - N)).reshape(ROWS, 128)
    total = pl.pallas_call(
        _k2_body,
        out_shape=jax.ShapeDtypeStruct((1,), jnp.float32),
        in_specs=[pl.BlockSpec((ROWS, 128), lambda: (0, 0))],
        out_specs=pl.BlockSpec(memory_space=pltpu.SMEM),
    )(wp)

    wt = wp.T
    s1t = pl.pallas_call(
        _k3_body,
        out_shape=jax.ShapeDtypeStruct((128, ROWS), jnp.float32),
        in_specs=[pl.BlockSpec((128, ROWS), lambda: (0, 0)),
                  pl.BlockSpec((1,), lambda: (0,))],
        out_specs=pl.BlockSpec((128, ROWS), lambda: (0, 0)),
    )(wt, total)

    tot1 = s1t[127, :VALID_ROWS]
    lvl2 = jnp.cumsum(tot1)
    off1 = jnp.pad(jnp.concatenate([jnp.zeros((1,), jnp.float32), lvl2[:-1]]),
                   (0, ROWS - VALID_ROWS)).reshape(1, ROWS)

    cdft, probs = pl.pallas_call(
        _k5_body,
        out_shape=[jax.ShapeDtypeStruct((128, ROWS), jnp.float32),
                   jax.ShapeDtypeStruct((N,), jnp.float32)],
        in_specs=[pl.BlockSpec((128, ROWS), lambda: (0, 0)),
                  pl.BlockSpec((1, ROWS), lambda: (0, 0)),
                  pl.BlockSpec((N,), lambda: (0,)),
                  pl.BlockSpec((1,), lambda: (0,))],
        out_specs=[pl.BlockSpec((128, ROWS), lambda: (0, 0)),
                   pl.BlockSpec((N,), lambda: (0,))],
    )(s1t, off1, w_eff, total)

    cdf = cdft.T.reshape(-1)[:N]
    table = cdf[15::16]
    cdf_rows = cdf.reshape(NT, 16)
    sz_bits = lax.bitcast_convert_type((jnp.int32(127) - levels) << 23,
                                       jnp.float32)
    packed = jnp.concatenate(
        [centers, probs[:, None], sz_bits[:, None],
         jnp.zeros((N, 11), jnp.float32)], axis=1)

    key = jax.random.key(SEED)
    ku, kn = jax.random.split(key)
    u = jax.random.uniform(ku, (NS,), dtype=jnp.float32)
    noise = jax.random.normal(kn, (NS, 3), dtype=jnp.float32)
    isz = jnp.broadcast_to(initial_size, (16,))

    px, py, pz, pr = _make_sc_kernel()(
        table, cdf_rows, u, packed,
        noise[:, 0], noise[:, 1], noise[:, 2], isz)
    return jnp.stack([px, py, pz, pr], axis=1)
